# baseline (speedup 1.0000x reference)
import functools

import jax
import jax.numpy as jnp
from jax import lax
from jax.experimental import pallas as pl
from jax.experimental.pallas import tpu as pltpu
from jax.experimental.pallas import tpu_sc as plsc

NVK = 40962
NFK = 81920
BC = 256
NW = 32
NVP = 41472

FC = 32
FPW = NFK // NW
NCH_A = FPW // FC

VC = 16
VPW = NVP // NW
NCH_B = VPW // VC

_MESH = plsc.VectorSubcoreMesh(core_axis_name="c", subcore_axis_name="s")
_PK = plsc.PackFormat.INTERLEAVED

RC = 24
NCH_P = VPW // RC


def _widx(w):
    return jnp.full((16,), w, dtype=jnp.float32)


@functools.partial(
    pl.kernel,
    out_type=jax.ShapeDtypeStruct((NVP, BC // 2), jnp.int32),
    mesh=_MESH,
    scratch_types=[
        pltpu.VMEM((2, RC, BC), jnp.float32),
        pltpu.VMEM((2, RC, BC // 2), jnp.int32),
        pltpu.SemaphoreType.DMA,
        pltpu.SemaphoreType.DMA,
        pltpu.SemaphoreType.DMA,
        pltpu.SemaphoreType.DMA,
    ],
    compiler_params=pltpu.CompilerParams(needs_layout_passes=False),
)
def _pk_kernel(xt_hbm, xi_hbm, in_v, out_v, gi0, gi1, go0, go1):
    wid = lax.axis_index("s") * 2 + lax.axis_index("c")
    base = wid * VPW
    gi = (gi0, gi1)
    go = (go0, go1)

    def in_issue(c, p):
        v0 = base + c * RC
        pltpu.async_copy(xt_hbm.at[pl.ds(v0, RC)], in_v.at[p], gi[p])

    def in_wait(c, p):
        v0 = base + c * RC
        pltpu.make_async_copy(xt_hbm.at[pl.ds(v0, RC)], in_v.at[p], gi[p]).wait()

    def out_issue(c, p):
        v0 = base + c * RC
        pltpu.async_copy(out_v.at[p], xi_hbm.at[pl.ds(v0, RC)], go[p])

    def out_wait(c, p):
        v0 = base + c * RC
        pltpu.make_async_copy(out_v.at[p], xi_hbm.at[pl.ds(v0, RC)], go[p]).wait()

    in_issue(0, 0)
    in_issue(1, 1)

    def body2(c2, carry):
        for p in (0, 1):
            c = 2 * c2 + p
            q = 1 - p
            in_wait(c, p)

            @pl.when(c >= 2)
            def _():
                out_wait(c, p)

            @plsc.parallel_loop(0, RC, unroll=4)
            def row_body(i):
                for cc in range(8):
                    a = in_v[p, i, pl.ds(32 * cc, 16)]
                    b = in_v[p, i, pl.ds(32 * cc + 16, 16)]
                    out_v[p, i, pl.ds(16 * cc, 16)] = plsc.bitcast(
                        plsc.pack(a, b, format=_PK), jnp.int32)

            out_issue(c, p)

            @pl.when(c + 2 < NCH_P)
            def _():
                in_issue(c + 2, p)
        return carry

    lax.fori_loop(0, NCH_P // 2, body2, 0)
    out_wait(NCH_P - 2, 0)
    out_wait(NCH_P - 1, 1)


@functools.partial(
    pl.kernel,
    out_type=jax.ShapeDtypeStruct((NFK, BC), jnp.int32),
    mesh=_MESH,
    scratch_types=[
        pltpu.VMEM((2, FC * 3), jnp.int32),
        pltpu.VMEM((2, FC * 3), jnp.int32),
        pltpu.VMEM((2, FC * 3), jnp.int32),
        pltpu.VMEM((2, FC * 3 + 16), jnp.float32),
        pltpu.VMEM((2, FC * 3 + 16), jnp.float32),
        pltpu.VMEM((2, FC * 3 + 16), jnp.float32),
        pltpu.VMEM((2, FC * 3 + 16), jnp.float32),
        pltpu.VMEM((2, FC * 3 + 16), jnp.float32),
        pltpu.VMEM((2, FC * 3, BC // 2), jnp.int32),
        pltpu.VMEM((2, FC * 3, BC // 2), jnp.int32),
        pltpu.VMEM((2, FC * 3, BC // 2), jnp.int32),
        pltpu.VMEM((2, FC, BC), jnp.int32),
        pltpu.SemaphoreType.DMA,
        pltpu.SemaphoreType.DMA,
        pltpu.SemaphoreType.DMA,
        pltpu.SemaphoreType.DMA,
        pltpu.SemaphoreType.DMA,
        pltpu.SemaphoreType.DMA,
    ],
    compiler_params=pltpu.CompilerParams(needs_layout_passes=False),
)
def _gf_kernel(xt_hbm, gc_hbm, gv_hbm, ew_hbm, ns_hbm, gf_hbm,
               gc0_v, gc1_v, gc2_v, gv0_v, gv1_v, gv2_v, ew_v, ns_v,
               r0_v, r1_v, r2_v, out_v,
               ss0, ss1, gs0, gs1, os0, os1):
    wid = lax.axis_index("s") * 2 + lax.axis_index("c")
    base = wid * FPW
    ss = (ss0, ss1)
    gs = (gs0, gs1)
    osm = (os0, os1)
    gci = (gc0_v, gc1_v, gc2_v)
    gvv = (gv0_v, gv1_v, gv2_v)
    rows = (r0_v, r1_v, r2_v)

    def slab_issue(c, p):
        f0 = base + c * FC
        for k in range(3):
            pltpu.async_copy(gc_hbm.at[pl.ds((k * NFK + f0) * 3, FC * 3)],
                             gci[k].at[p], ss[p])
            pltpu.async_copy(gv_hbm.at[pl.ds((k * NFK + f0) * 3, FC * 3 + 16)],
                             gvv[k].at[p], ss[p])
        pltpu.async_copy(ew_hbm.at[pl.ds(f0 * 3, FC * 3 + 16)], ew_v.at[p], ss[p])
        pltpu.async_copy(ns_hbm.at[pl.ds(f0 * 3, FC * 3 + 16)], ns_v.at[p], ss[p])

    def slab_wait(c, p):
        f0 = base + c * FC
        for k in range(3):
            pltpu.make_async_copy(gc_hbm.at[pl.ds((k * NFK + f0) * 3, FC * 3)],
                                  gci[k].at[p], ss[p]).wait()
            pltpu.make_async_copy(gv_hbm.at[pl.ds((k * NFK + f0) * 3, FC * 3 + 16)],
                                  gvv[k].at[p], ss[p]).wait()
        pltpu.make_async_copy(ew_hbm.at[pl.ds(f0 * 3, FC * 3 + 16)], ew_v.at[p], ss[p]).wait()
        pltpu.make_async_copy(ns_hbm.at[pl.ds(f0 * 3, FC * 3 + 16)], ns_v.at[p], ss[p]).wait()

    def gather_issue(p):
        for k in range(3):
            pltpu.async_copy(xt_hbm.at[gci[k].at[p]], rows[k].at[p], gs[p])

    def gather_wait(p):
        for k in range(3):
            pltpu.make_async_copy(xt_hbm.at[gci[k].at[p]], rows[k].at[p], gs[p]).wait()

    def out_issue(c, p):
        f0 = base + c * FC
        pltpu.async_copy(out_v.at[p], gf_hbm.at[pl.ds(f0, FC)], osm[p])

    def out_wait(c, p):
        f0 = base + c * FC
        pltpu.make_async_copy(out_v.at[p], gf_hbm.at[pl.ds(f0, FC)], osm[p]).wait()

    slab_issue(0, 0)
    slab_wait(0, 0)
    gather_issue(0)
    slab_issue(1, 1)

    def body2(c2, carry):
        for p in (0, 1):
            c = 2 * c2 + p
            q = 1 - p
            gather_wait(p)

            @pl.when(c + 1 < NCH_A)
            def _():
                slab_wait(c + 1, q)
                gather_issue(q)

            @pl.when(c >= 2)
            def _():
                out_wait(c, p)

            @plsc.parallel_loop(0, FC, unroll=4)
            def face_body(i):
                ewr = ew_v[p, pl.ds(3 * i, 16)]
                nsr = ns_v[p, pl.ds(3 * i, 16)]
                we = []
                wn = []
                for k in range(3):
                    gvr = gvv[k][p, pl.ds(3 * i, 16)]
                    for j in range(3):
                        we.append(_widx(gvr[j] * ewr[k]))
                        wn.append(_widx(gvr[j] * nsr[k]))
                z = jnp.zeros((16,), jnp.float32)
                for cc in range(8):
                    ea, eb, na, nb = [z] * 3, [z] * 3, [z] * 3, [z] * 3
                    for k in range(3):
                        for j in range(3):
                            rw = rows[k][p, 3 * i + j, pl.ds(16 * cc, 16)]
                            ra, rb = plsc.unpack(plsc.bitcast(rw, jnp.bfloat16), format=_PK)
                            w = 3 * k + j
                            ea[k] = ea[k] + we[w] * ra
                            eb[k] = eb[k] + we[w] * rb
                            na[k] = na[k] + wn[w] * ra
                            nb[k] = nb[k] + wn[w] * rb
                    out_v[p, i, pl.ds(16 * cc, 16)] = plsc.bitcast(
                        plsc.pack(ea[0] + ea[1] + ea[2], eb[0] + eb[1] + eb[2],
                                  format=_PK), jnp.int32)
                    out_v[p, i, pl.ds(BC // 2 + 16 * cc, 16)] = plsc.bitcast(
                        plsc.pack(na[0] + na[1] + na[2], nb[0] + nb[1] + nb[2],
                                  format=_PK), jnp.int32)

            out_issue(c, p)

            @pl.when(c + 2 < NCH_A)
            def _():
                slab_issue(c + 2, p)
        return carry

    lax.fori_loop(0, NCH_A // 2, body2, 0)
    out_wait(NCH_A - 2, 0)
    out_wait(NCH_A - 1, 1)


@functools.partial(
    pl.kernel,
    out_type=(
        jax.ShapeDtypeStruct((NVP, BC), jnp.float32),
        jax.ShapeDtypeStruct((NVP, BC), jnp.float32),
        jax.ShapeDtypeStruct((NVP, BC), jnp.float32),
    ),
    mesh=_MESH,
    scratch_types=[
        pltpu.VMEM((2, VC * 7), jnp.int32),
        pltpu.VMEM((2, VC * 6), jnp.int32),
        pltpu.VMEM((2, VC, 16), jnp.float32),
        pltpu.VMEM((2, VC * 7, BC // 2), jnp.int32),
        pltpu.VMEM((2, VC * 6, BC), jnp.int32),
        pltpu.VMEM((2, VC, BC), jnp.float32),
        pltpu.VMEM((2, VC, BC), jnp.float32),
        pltpu.VMEM((2, VC, BC), jnp.float32),
        pltpu.SemaphoreType.DMA,
        pltpu.SemaphoreType.DMA,
        pltpu.SemaphoreType.DMA,
        pltpu.SemaphoreType.DMA,
        pltpu.SemaphoreType.DMA,
        pltpu.SemaphoreType.DMA,
    ],
    compiler_params=pltpu.CompilerParams(needs_layout_passes=False),
)
def _lv_kernel(xt_hbm, gf_hbm, li_hbm, fi_hbm, wb_hbm,
               lap_hbm, gve_hbm, gvn_hbm,
               li_v, fi_v, wb_v, rx_v, rg_v, ol_v, oe_v, on_v,
               ss0, ss1, gs0, gs1, os0, os1):
    wid = lax.axis_index("s") * 2 + lax.axis_index("c")
    base = wid * VPW
    ss = (ss0, ss1)
    gs = (gs0, gs1)
    osm = (os0, os1)

    def slab_issue(c, p):
        v0 = base + c * VC
        pltpu.async_copy(li_hbm.at[pl.ds(v0 * 7, VC * 7)], li_v.at[p], ss[p])
        pltpu.async_copy(fi_hbm.at[pl.ds(v0 * 6, VC * 6)], fi_v.at[p], ss[p])
        pltpu.async_copy(wb_hbm.at[pl.ds(v0, VC)], wb_v.at[p], ss[p])

    def slab_wait(c, p):
        v0 = base + c * VC
        pltpu.make_async_copy(li_hbm.at[pl.ds(v0 * 7, VC * 7)], li_v.at[p], ss[p]).wait()
        pltpu.make_async_copy(fi_hbm.at[pl.ds(v0 * 6, VC * 6)], fi_v.at[p], ss[p]).wait()
        pltpu.make_async_copy(wb_hbm.at[pl.ds(v0, VC)], wb_v.at[p], ss[p]).wait()

    def gather_issue(p):
        pltpu.async_copy(xt_hbm.at[li_v.at[p]], rx_v.at[p], gs[p])
        pltpu.async_copy(gf_hbm.at[fi_v.at[p]], rg_v.at[p], gs[p])

    def gather_wait(p):
        pltpu.make_async_copy(xt_hbm.at[li_v.at[p]], rx_v.at[p], gs[p]).wait()
        pltpu.make_async_copy(gf_hbm.at[fi_v.at[p]], rg_v.at[p], gs[p]).wait()

    def out_issue(c, p):
        v0 = base + c * VC
        pltpu.async_copy(ol_v.at[p], lap_hbm.at[pl.ds(v0, VC)], osm[p])
        pltpu.async_copy(oe_v.at[p], gve_hbm.at[pl.ds(v0, VC)], osm[p])
        pltpu.async_copy(on_v.at[p], gvn_hbm.at[pl.ds(v0, VC)], osm[p])

    def out_wait(c, p):
        v0 = base + c * VC
        pltpu.make_async_copy(ol_v.at[p], lap_hbm.at[pl.ds(v0, VC)], osm[p]).wait()
        pltpu.make_async_copy(oe_v.at[p], gve_hbm.at[pl.ds(v0, VC)], osm[p]).wait()
        pltpu.make_async_copy(on_v.at[p], gvn_hbm.at[pl.ds(v0, VC)], osm[p]).wait()

    slab_issue(0, 0)
    slab_wait(0, 0)
    gather_issue(0)
    slab_issue(1, 1)

    def do_chunk(c, p, q):
            gather_wait(p)

            @pl.when(c + 1 < NCH_B)
            def _():
                slab_wait(c + 1, q)
                gather_issue(q)

            @pl.when(c >= 2)
            def _():
                out_wait(c, p)

            @plsc.parallel_loop(0, VC, unroll=4)
            def vert_body(i):
                wrow = wb_v[p, i, :]
                z = jnp.zeros((16,), jnp.float32)
                wl = [_widx(wrow[j]) for j in range(7)]
                for cc in range(8):
                    la, lb = [z] * 2, [z] * 2
                    for j in range(7):
                        rw = rx_v[p, 7 * i + j, pl.ds(16 * cc, 16)]
                        ra, rb = plsc.unpack(plsc.bitcast(rw, jnp.bfloat16), format=_PK)
                        la[j % 2] = la[j % 2] + wl[j] * ra
                        lb[j % 2] = lb[j % 2] + wl[j] * rb
                    ol_v[p, i, pl.ds(32 * cc, 16)] = la[0] + la[1]
                    ol_v[p, i, pl.ds(32 * cc + 16, 16)] = lb[0] + lb[1]
                wf = [_widx(wrow[7 + j]) for j in range(6)]
                for cc in range(8):
                    ea, eb, na, nb = [z] * 2, [z] * 2, [z] * 2, [z] * 2
                    for j in range(6):
                        rw = rg_v[p, 6 * i + j, pl.ds(16 * cc, 16)]
                        ra, rb = plsc.unpack(plsc.bitcast(rw, jnp.bfloat16), format=_PK)
                        sw = rg_v[p, 6 * i + j, pl.ds(BC // 2 + 16 * cc, 16)]
                        sa, sb = plsc.unpack(plsc.bitcast(sw, jnp.bfloat16), format=_PK)
                        h = j % 2
                        ea[h] = ea[h] + wf[j] * ra
                        eb[h] = eb[h] + wf[j] * rb
                        na[h] = na[h] + wf[j] * sa
                        nb[h] = nb[h] + wf[j] * sb
                    oe_v[p, i, pl.ds(32 * cc, 16)] = ea[0] + ea[1]
                    oe_v[p, i, pl.ds(32 * cc + 16, 16)] = eb[0] + eb[1]
                    on_v[p, i, pl.ds(32 * cc, 16)] = na[0] + na[1]
                    on_v[p, i, pl.ds(32 * cc + 16, 16)] = nb[0] + nb[1]

            out_issue(c, p)

            @pl.when(c + 2 < NCH_B)
            def _():
                slab_issue(c + 2, p)

    def body2(c2, carry):
        do_chunk(2 * c2, 0, 1)
        do_chunk(2 * c2 + 1, 1, 0)
        return carry

    lax.fori_loop(0, NCH_B // 2, body2, 0)
    do_chunk(NCH_B - 1, 0, 1)
    out_wait(NCH_B - 2, 1)
    out_wait(NCH_B - 1, 0)


def _tin_body(x_ref, o_ref):
    o_ref[...] = x_ref[...].T


_tin = pl.pallas_call(
    _tin_body,
    grid=(NVP // 512,),
    in_specs=[pl.BlockSpec((BC, 512), lambda i: (0, i))],
    out_specs=pl.BlockSpec((512, BC), lambda i: (i, 0)),
    out_shape=jax.ShapeDtypeStruct((NVP, BC), jnp.float32),
)


def _mmt_body(x_ref, l_ref, e_ref, n_ref, cs_ref, o_ref):
    cs = cs_ref[...]
    feats = (x_ref, l_ref, e_ref, n_ref)
    cols = []
    for b in range(4):
        acc = jnp.zeros((512, 64), jnp.float32)
        for k in range(4):
            acc += jnp.dot(feats[k][:, 64 * b:64 * (b + 1)], cs[k],
                           preferred_element_type=jnp.float32)
        cols.append(acc)
    o_ref[...] = jnp.concatenate(cols, axis=1).T


_mmt = pl.pallas_call(
    _mmt_body,
    grid=(NVP // 512,),
    in_specs=[
        pl.BlockSpec((512, BC), lambda i: (i, 0)),
        pl.BlockSpec((512, BC), lambda i: (i, 0)),
        pl.BlockSpec((512, BC), lambda i: (i, 0)),
        pl.BlockSpec((512, BC), lambda i: (i, 0)),
        pl.BlockSpec((4, 64, 64), lambda i: (0, 0, 0)),
    ],
    out_specs=pl.BlockSpec((BC, 512), lambda i: (0, i)),
    out_shape=jax.ShapeDtypeStruct((BC, NVK), jnp.float32),
)


def kernel(input, G_vals, L_vals, F2V_vals, NS, EW, coeffs,
           G_rows, G_cols, L_rows, L_cols, F2V_rows, F2V_cols):
    pad = NVP - NVK
    li7 = jnp.pad(L_cols, (0, pad * 7))
    fi6 = jnp.pad(F2V_cols, (0, pad * 6))
    wmb = jnp.pad(
        jnp.concatenate([L_vals.reshape(NVK, 7), F2V_vals.reshape(NVK, 6),
                         jnp.zeros((NVK, 3), jnp.float32)], axis=1),
        ((0, pad), (0, 0)))
    cs = jnp.stack([coeffs[k::4] for k in range(4)])

    xtp = _tin(input.reshape(BC, NVK))
    xti = _pk_kernel(xtp)
    gvp = jnp.pad(G_vals, (0, 64))
    ewp = jnp.pad(EW.reshape(-1), (0, 64))
    nsp = jnp.pad(NS.reshape(-1), (0, 64))
    gf = _gf_kernel(xti, G_cols, gvp, ewp, nsp)
    lap, gve, gvn = _lv_kernel(xti, gf, li7, fi6, wmb)

    return _mmt(xtp, lap, gve, gvn, cs).reshape(4, 64, NVK)


# speedup vs baseline: 1.4254x; 1.4254x over previous
import functools

import jax
import jax.numpy as jnp
from jax import lax
from jax.experimental import pallas as pl
from jax.experimental.pallas import tpu as pltpu
from jax.experimental.pallas import tpu_sc as plsc

NVK = 40962
NFK = 81920
BC = 256
NW = 32
NVP = 41472

FC = 32
FPW = NFK // NW
NCH_A = FPW // FC

VC = 16
VPW = NVP // NW
NCH_B = VPW // VC

_MESH = plsc.VectorSubcoreMesh(core_axis_name="c", subcore_axis_name="s")
_PK = plsc.PackFormat.INTERLEAVED

RC = 24
NCH_P = VPW // RC


def _widx(w):
    return jnp.full((16,), w, dtype=jnp.float32)


@functools.partial(
    pl.kernel,
    out_type=jax.ShapeDtypeStruct((NVP, BC // 2), jnp.int32),
    mesh=_MESH,
    scratch_types=[
        pltpu.VMEM((2, RC, BC), jnp.float32),
        pltpu.VMEM((2, RC, BC // 2), jnp.int32),
        pltpu.SemaphoreType.DMA,
        pltpu.SemaphoreType.DMA,
        pltpu.SemaphoreType.DMA,
        pltpu.SemaphoreType.DMA,
    ],
    compiler_params=pltpu.CompilerParams(needs_layout_passes=False),
)
def _pk_kernel(xt_hbm, xi_hbm, in_v, out_v, gi0, gi1, go0, go1):
    wid = lax.axis_index("s") * 2 + lax.axis_index("c")
    base = wid * VPW
    gi = (gi0, gi1)
    go = (go0, go1)

    def in_issue(c, p):
        v0 = base + c * RC
        pltpu.async_copy(xt_hbm.at[pl.ds(v0, RC)], in_v.at[p], gi[p])

    def in_wait(c, p):
        v0 = base + c * RC
        pltpu.make_async_copy(xt_hbm.at[pl.ds(v0, RC)], in_v.at[p], gi[p]).wait()

    def out_issue(c, p):
        v0 = base + c * RC
        pltpu.async_copy(out_v.at[p], xi_hbm.at[pl.ds(v0, RC)], go[p])

    def out_wait(c, p):
        v0 = base + c * RC
        pltpu.make_async_copy(out_v.at[p], xi_hbm.at[pl.ds(v0, RC)], go[p]).wait()

    in_issue(0, 0)
    in_issue(1, 1)

    def body2(c2, carry):
        for p in (0, 1):
            c = 2 * c2 + p
            q = 1 - p
            in_wait(c, p)

            @pl.when(c >= 2)
            def _():
                out_wait(c, p)

            @plsc.parallel_loop(0, RC, unroll=4)
            def row_body(i):
                for cc in range(8):
                    a = in_v[p, i, pl.ds(32 * cc, 16)]
                    b = in_v[p, i, pl.ds(32 * cc + 16, 16)]
                    out_v[p, i, pl.ds(16 * cc, 16)] = plsc.bitcast(
                        plsc.pack(a, b, format=_PK), jnp.int32)

            out_issue(c, p)

            @pl.when(c + 2 < NCH_P)
            def _():
                in_issue(c + 2, p)
        return carry

    lax.fori_loop(0, NCH_P // 2, body2, 0)
    out_wait(NCH_P - 2, 0)
    out_wait(NCH_P - 1, 1)


@functools.partial(
    pl.kernel,
    out_type=jax.ShapeDtypeStruct((NFK, BC), jnp.int32),
    mesh=_MESH,
    scratch_types=[
        pltpu.VMEM((2, FC * 3), jnp.int32),
        pltpu.VMEM((2, FC * 3), jnp.int32),
        pltpu.VMEM((2, FC * 3), jnp.int32),
        pltpu.VMEM((2, FC * 3 + 16), jnp.float32),
        pltpu.VMEM((2, FC * 3 + 16), jnp.float32),
        pltpu.VMEM((2, FC * 3 + 16), jnp.float32),
        pltpu.VMEM((2, FC * 3 + 16), jnp.float32),
        pltpu.VMEM((2, FC * 3 + 16), jnp.float32),
        pltpu.VMEM((2, FC * 3, BC // 2), jnp.int32),
        pltpu.VMEM((2, FC * 3, BC // 2), jnp.int32),
        pltpu.VMEM((2, FC * 3, BC // 2), jnp.int32),
        pltpu.VMEM((2, FC, BC), jnp.int32),
        pltpu.SemaphoreType.DMA,
        pltpu.SemaphoreType.DMA,
        pltpu.SemaphoreType.DMA,
        pltpu.SemaphoreType.DMA,
        pltpu.SemaphoreType.DMA,
        pltpu.SemaphoreType.DMA,
    ],
    compiler_params=pltpu.CompilerParams(needs_layout_passes=False),
)
def _gf_kernel(xt_hbm, gc_hbm, gv_hbm, ew_hbm, ns_hbm, gf_hbm,
               gc0_v, gc1_v, gc2_v, gv0_v, gv1_v, gv2_v, ew_v, ns_v,
               r0_v, r1_v, r2_v, out_v,
               ss0, ss1, gs0, gs1, os0, os1):
    wid = lax.axis_index("s") * 2 + lax.axis_index("c")
    base = wid * FPW
    ss = (ss0, ss1)
    gs = (gs0, gs1)
    osm = (os0, os1)
    gci = (gc0_v, gc1_v, gc2_v)
    gvv = (gv0_v, gv1_v, gv2_v)
    rows = (r0_v, r1_v, r2_v)

    def slab_issue(c, p):
        f0 = base + c * FC
        for k in range(3):
            pltpu.async_copy(gc_hbm.at[pl.ds((k * NFK + f0) * 3, FC * 3)],
                             gci[k].at[p], ss[p])
            pltpu.async_copy(gv_hbm.at[pl.ds((k * NFK + f0) * 3, FC * 3 + 16)],
                             gvv[k].at[p], ss[p])
        pltpu.async_copy(ew_hbm.at[pl.ds(f0 * 3, FC * 3 + 16)], ew_v.at[p], ss[p])
        pltpu.async_copy(ns_hbm.at[pl.ds(f0 * 3, FC * 3 + 16)], ns_v.at[p], ss[p])

    def slab_wait(c, p):
        f0 = base + c * FC
        for k in range(3):
            pltpu.make_async_copy(gc_hbm.at[pl.ds((k * NFK + f0) * 3, FC * 3)],
                                  gci[k].at[p], ss[p]).wait()
            pltpu.make_async_copy(gv_hbm.at[pl.ds((k * NFK + f0) * 3, FC * 3 + 16)],
                                  gvv[k].at[p], ss[p]).wait()
        pltpu.make_async_copy(ew_hbm.at[pl.ds(f0 * 3, FC * 3 + 16)], ew_v.at[p], ss[p]).wait()
        pltpu.make_async_copy(ns_hbm.at[pl.ds(f0 * 3, FC * 3 + 16)], ns_v.at[p], ss[p]).wait()

    def gather_issue(p):
        for k in range(3):
            pltpu.async_copy(xt_hbm.at[gci[k].at[p]], rows[k].at[p], gs[p])

    def gather_wait(p):
        for k in range(3):
            pltpu.make_async_copy(xt_hbm.at[gci[k].at[p]], rows[k].at[p], gs[p]).wait()

    def out_issue(c, p):
        f0 = base + c * FC
        pltpu.async_copy(out_v.at[p], gf_hbm.at[pl.ds(f0, FC)], osm[p])

    def out_wait(c, p):
        f0 = base + c * FC
        pltpu.make_async_copy(out_v.at[p], gf_hbm.at[pl.ds(f0, FC)], osm[p]).wait()

    slab_issue(0, 0)
    slab_wait(0, 0)
    gather_issue(0)
    slab_issue(1, 1)

    def body2(c2, carry):
        for p in (0, 1):
            c = 2 * c2 + p
            q = 1 - p
            gather_wait(p)

            @pl.when(c + 1 < NCH_A)
            def _():
                slab_wait(c + 1, q)
                gather_issue(q)

            @pl.when(c >= 2)
            def _():
                out_wait(c, p)

            @plsc.parallel_loop(0, FC, unroll=2)
            def face_body(i):
                ewr = ew_v[p, pl.ds(3 * i, 16)]
                nsr = ns_v[p, pl.ds(3 * i, 16)]
                we = []
                wn = []
                for k in range(3):
                    gvr = gvv[k][p, pl.ds(3 * i, 16)]
                    for j in range(3):
                        we.append(_widx(gvr[j] * ewr[k]))
                        wn.append(_widx(gvr[j] * nsr[k]))
                z = jnp.zeros((16,), jnp.float32)
                for cc in range(8):
                    ea, eb, na, nb = [z] * 3, [z] * 3, [z] * 3, [z] * 3
                    for k in range(3):
                        for j in range(3):
                            rw = rows[k][p, 3 * i + j, pl.ds(16 * cc, 16)]
                            ra, rb = plsc.unpack(plsc.bitcast(rw, jnp.bfloat16), format=_PK)
                            w = 3 * k + j
                            ea[k] = ea[k] + we[w] * ra
                            eb[k] = eb[k] + we[w] * rb
                            na[k] = na[k] + wn[w] * ra
                            nb[k] = nb[k] + wn[w] * rb
                    out_v[p, i, pl.ds(16 * cc, 16)] = plsc.bitcast(
                        plsc.pack(ea[0] + ea[1] + ea[2], eb[0] + eb[1] + eb[2],
                                  format=_PK), jnp.int32)
                    out_v[p, i, pl.ds(BC // 2 + 16 * cc, 16)] = plsc.bitcast(
                        plsc.pack(na[0] + na[1] + na[2], nb[0] + nb[1] + nb[2],
                                  format=_PK), jnp.int32)

            out_issue(c, p)

            @pl.when(c + 2 < NCH_A)
            def _():
                slab_issue(c + 2, p)
        return carry

    lax.fori_loop(0, NCH_A // 2, body2, 0)
    out_wait(NCH_A - 2, 0)
    out_wait(NCH_A - 1, 1)


@functools.partial(
    pl.kernel,
    out_type=(
        jax.ShapeDtypeStruct((NVP, BC), jnp.float32),
        jax.ShapeDtypeStruct((NVP, BC), jnp.float32),
        jax.ShapeDtypeStruct((NVP, BC), jnp.float32),
    ),
    mesh=_MESH,
    scratch_types=[
        pltpu.VMEM((2, VC * 7), jnp.int32),
        pltpu.VMEM((2, VC * 6), jnp.int32),
        pltpu.VMEM((2, VC, 16), jnp.float32),
        pltpu.VMEM((2, VC * 7, BC // 2), jnp.int32),
        pltpu.VMEM((2, VC * 6, BC), jnp.int32),
        pltpu.VMEM((2, VC, BC), jnp.float32),
        pltpu.VMEM((2, VC, BC), jnp.float32),
        pltpu.VMEM((2, VC, BC), jnp.float32),
        pltpu.SemaphoreType.DMA,
        pltpu.SemaphoreType.DMA,
        pltpu.SemaphoreType.DMA,
        pltpu.SemaphoreType.DMA,
        pltpu.SemaphoreType.DMA,
        pltpu.SemaphoreType.DMA,
    ],
    compiler_params=pltpu.CompilerParams(needs_layout_passes=False),
)
def _lv_kernel(xt_hbm, gf_hbm, li_hbm, fi_hbm, wb_hbm,
               lap_hbm, gve_hbm, gvn_hbm,
               li_v, fi_v, wb_v, rx_v, rg_v, ol_v, oe_v, on_v,
               ss0, ss1, gs0, gs1, os0, os1):
    wid = lax.axis_index("s") * 2 + lax.axis_index("c")
    base = wid * VPW
    ss = (ss0, ss1)
    gs = (gs0, gs1)
    osm = (os0, os1)

    def slab_issue(c, p):
        v0 = base + c * VC
        pltpu.async_copy(li_hbm.at[pl.ds(v0 * 7, VC * 7)], li_v.at[p], ss[p])
        pltpu.async_copy(fi_hbm.at[pl.ds(v0 * 6, VC * 6)], fi_v.at[p], ss[p])
        pltpu.async_copy(wb_hbm.at[pl.ds(v0, VC)], wb_v.at[p], ss[p])

    def slab_wait(c, p):
        v0 = base + c * VC
        pltpu.make_async_copy(li_hbm.at[pl.ds(v0 * 7, VC * 7)], li_v.at[p], ss[p]).wait()
        pltpu.make_async_copy(fi_hbm.at[pl.ds(v0 * 6, VC * 6)], fi_v.at[p], ss[p]).wait()
        pltpu.make_async_copy(wb_hbm.at[pl.ds(v0, VC)], wb_v.at[p], ss[p]).wait()

    def gather_issue(p):
        pltpu.async_copy(xt_hbm.at[li_v.at[p]], rx_v.at[p], gs[p])
        pltpu.async_copy(gf_hbm.at[fi_v.at[p]], rg_v.at[p], gs[p])

    def gather_wait(p):
        pltpu.make_async_copy(xt_hbm.at[li_v.at[p]], rx_v.at[p], gs[p]).wait()
        pltpu.make_async_copy(gf_hbm.at[fi_v.at[p]], rg_v.at[p], gs[p]).wait()

    def out_issue(c, p):
        v0 = base + c * VC
        pltpu.async_copy(ol_v.at[p], lap_hbm.at[pl.ds(v0, VC)], osm[p])
        pltpu.async_copy(oe_v.at[p], gve_hbm.at[pl.ds(v0, VC)], osm[p])
        pltpu.async_copy(on_v.at[p], gvn_hbm.at[pl.ds(v0, VC)], osm[p])

    def out_wait(c, p):
        v0 = base + c * VC
        pltpu.make_async_copy(ol_v.at[p], lap_hbm.at[pl.ds(v0, VC)], osm[p]).wait()
        pltpu.make_async_copy(oe_v.at[p], gve_hbm.at[pl.ds(v0, VC)], osm[p]).wait()
        pltpu.make_async_copy(on_v.at[p], gvn_hbm.at[pl.ds(v0, VC)], osm[p]).wait()

    slab_issue(0, 0)
    slab_wait(0, 0)
    gather_issue(0)
    slab_issue(1, 1)

    def do_chunk(c, p, q):
            gather_wait(p)

            @pl.when(c + 1 < NCH_B)
            def _():
                slab_wait(c + 1, q)
                gather_issue(q)

            @pl.when(c >= 2)
            def _():
                out_wait(c, p)

            @plsc.parallel_loop(0, VC, unroll=2)
            def vert_body(i):
                wrow = wb_v[p, i, :]
                z = jnp.zeros((16,), jnp.float32)
                wl = [_widx(wrow[j]) for j in range(7)]
                for cc in range(8):
                    la, lb = [z] * 2, [z] * 2
                    for j in range(7):
                        rw = rx_v[p, 7 * i + j, pl.ds(16 * cc, 16)]
                        ra, rb = plsc.unpack(plsc.bitcast(rw, jnp.bfloat16), format=_PK)
                        la[j % 2] = la[j % 2] + wl[j] * ra
                        lb[j % 2] = lb[j % 2] + wl[j] * rb
                    ol_v[p, i, pl.ds(32 * cc, 16)] = la[0] + la[1]
                    ol_v[p, i, pl.ds(32 * cc + 16, 16)] = lb[0] + lb[1]
                wf = [_widx(wrow[7 + j]) for j in range(6)]
                for cc in range(8):
                    ea, eb, na, nb = [z] * 2, [z] * 2, [z] * 2, [z] * 2
                    for j in range(6):
                        rw = rg_v[p, 6 * i + j, pl.ds(16 * cc, 16)]
                        ra, rb = plsc.unpack(plsc.bitcast(rw, jnp.bfloat16), format=_PK)
                        sw = rg_v[p, 6 * i + j, pl.ds(BC // 2 + 16 * cc, 16)]
                        sa, sb = plsc.unpack(plsc.bitcast(sw, jnp.bfloat16), format=_PK)
                        h = j % 2
                        ea[h] = ea[h] + wf[j] * ra
                        eb[h] = eb[h] + wf[j] * rb
                        na[h] = na[h] + wf[j] * sa
                        nb[h] = nb[h] + wf[j] * sb
                    oe_v[p, i, pl.ds(32 * cc, 16)] = ea[0] + ea[1]
                    oe_v[p, i, pl.ds(32 * cc + 16, 16)] = eb[0] + eb[1]
                    on_v[p, i, pl.ds(32 * cc, 16)] = na[0] + na[1]
                    on_v[p, i, pl.ds(32 * cc + 16, 16)] = nb[0] + nb[1]

            out_issue(c, p)

            @pl.when(c + 2 < NCH_B)
            def _():
                slab_issue(c + 2, p)

    def body2(c2, carry):
        do_chunk(2 * c2, 0, 1)
        do_chunk(2 * c2 + 1, 1, 0)
        return carry

    lax.fori_loop(0, NCH_B // 2, body2, 0)
    do_chunk(NCH_B - 1, 0, 1)
    out_wait(NCH_B - 2, 1)
    out_wait(NCH_B - 1, 0)


def _tin_body(x_ref, o_ref):
    o_ref[...] = x_ref[...].T


_tin = pl.pallas_call(
    _tin_body,
    grid=(NVP // 512,),
    in_specs=[pl.BlockSpec((BC, 512), lambda i: (0, i))],
    out_specs=pl.BlockSpec((512, BC), lambda i: (i, 0)),
    out_shape=jax.ShapeDtypeStruct((NVP, BC), jnp.float32),
)


def _mmt_body(x_ref, l_ref, e_ref, n_ref, cs_ref, o_ref):
    cs = cs_ref[...]
    feats = (x_ref, l_ref, e_ref, n_ref)
    cols = []
    for b in range(4):
        acc = jnp.zeros((512, 64), jnp.float32)
        for k in range(4):
            acc += jnp.dot(feats[k][:, 64 * b:64 * (b + 1)], cs[k],
                           preferred_element_type=jnp.float32)
        cols.append(acc)
    o_ref[...] = jnp.concatenate(cols, axis=1).T


_mmt = pl.pallas_call(
    _mmt_body,
    grid=(NVP // 512,),
    in_specs=[
        pl.BlockSpec((512, BC), lambda i: (i, 0)),
        pl.BlockSpec((512, BC), lambda i: (i, 0)),
        pl.BlockSpec((512, BC), lambda i: (i, 0)),
        pl.BlockSpec((512, BC), lambda i: (i, 0)),
        pl.BlockSpec((4, 64, 64), lambda i: (0, 0, 0)),
    ],
    out_specs=pl.BlockSpec((BC, 512), lambda i: (0, i)),
    out_shape=jax.ShapeDtypeStruct((BC, NVK), jnp.float32),
)


def kernel(input, G_vals, L_vals, F2V_vals, NS, EW, coeffs,
           G_rows, G_cols, L_rows, L_cols, F2V_rows, F2V_cols):
    pad = NVP - NVK
    li7 = jnp.pad(L_cols, (0, pad * 7))
    fi6 = jnp.pad(F2V_cols, (0, pad * 6))
    wmb = jnp.pad(
        jnp.concatenate([L_vals.reshape(NVK, 7), F2V_vals.reshape(NVK, 6),
                         jnp.zeros((NVK, 3), jnp.float32)], axis=1),
        ((0, pad), (0, 0)))
    cs = jnp.stack([coeffs[k::4] for k in range(4)])

    xtp = _tin(input.reshape(BC, NVK))
    xti = _pk_kernel(xtp)
    gvp = jnp.pad(G_vals, (0, 64))
    ewp = jnp.pad(EW.reshape(-1), (0, 64))
    nsp = jnp.pad(NS.reshape(-1), (0, 64))
    gf = _gf_kernel(xti, G_cols, gvp, ewp, nsp)
    lap, gve, gvn = _lv_kernel(xti, gf, li7, fi6, wmb)

    return _mmt(xtp, lap, gve, gvn, cs).reshape(4, 64, NVK)
